# baseline (speedup 1.0000x reference)
import functools

import jax
import jax.numpy as jnp
from jax import lax
from jax.experimental import pallas as pl
from jax.experimental.pallas import tpu as pltpu
from jax.experimental.pallas import tpu_sc as plsc

BATCH = 4096
WALK_NUM = 8
WALK_LEN = 8
DIM = 64
PATHS = BATCH * WALK_NUM

NUM_CORES = 2
NUM_SUBCORES = 16
NUM_WORKERS = NUM_CORES * NUM_SUBCORES
PATHS_PER_WORKER = PATHS // NUM_WORKERS
CHUNK = 32
NUM_CHUNKS = PATHS_PER_WORKER // CHUNK

N_CAS_STEPS = 4
N_USER_STEPS = 3
INV_STEPS = 1.0 / (N_CAS_STEPS + N_USER_STEPS)


def _sc_body(cas_idx_h, user_idx_h, pt_h, cas_tab_h, user_tab_h, tw_h,
             out_h,
             cas_idx_v, user_idx_v, pt_v, cas_rows_v, user_rows_v, out_v,
             tw_v, sem_cas, sem_user):
    wid = lax.axis_index("s") * NUM_CORES + lax.axis_index("c")

    pltpu.sync_copy(tw_h, tw_v)
    tw4 = [tw_v[pl.ds(16 * d, 16)] * (N_CAS_STEPS * INV_STEPS)
           for d in range(DIM // 16)]

    def chunk_step(k, carry):
        pltpu.sync_copy(cas_idx_h.at[wid, k], cas_idx_v)
        pltpu.sync_copy(user_idx_h.at[wid, k], user_idx_v)
        pltpu.sync_copy(pt_h.at[wid, k], pt_v)
        cp_cas = pltpu.async_copy(cas_tab_h.at[cas_idx_v], cas_rows_v,
                                  sem_cas)
        cp_user = pltpu.async_copy(user_tab_h.at[user_idx_v], user_rows_v,
                                   sem_user)
        cp_cas.wait()
        cp_user.wait()
        for c in range(CHUNK):
            ptv = pt_v[c, :]
            for d in range(DIM // 16):
                sl = pl.ds(16 * d, 16)
                acc = cas_rows_v[c * N_CAS_STEPS + 0, sl]
                acc = acc + cas_rows_v[c * N_CAS_STEPS + 1, sl]
                acc = acc + cas_rows_v[c * N_CAS_STEPS + 2, sl]
                acc = acc + cas_rows_v[c * N_CAS_STEPS + 3, sl]
                acc = acc + user_rows_v[c * N_USER_STEPS + 0, sl]
                acc = acc + user_rows_v[c * N_USER_STEPS + 1, sl]
                acc = acc + user_rows_v[c * N_USER_STEPS + 2, sl]
                out_v[c, sl] = acc * INV_STEPS + ptv * tw4[d]
        pltpu.sync_copy(
            out_v, out_h.at[pl.ds(wid * PATHS_PER_WORKER + k * CHUNK, CHUNK)])
        return carry

    lax.fori_loop(0, NUM_CHUNKS, chunk_step, 0)


_sc_call = functools.partial(
    pl.kernel,
    out_type=jax.ShapeDtypeStruct((PATHS, DIM), jnp.float32),
    mesh=plsc.VectorSubcoreMesh(core_axis_name="c", subcore_axis_name="s",
                                num_cores=NUM_CORES,
                                num_subcores=NUM_SUBCORES),
    compiler_params=pltpu.CompilerParams(use_tc_tiling_on_sc=False),
    scratch_types=[
        pltpu.VMEM((CHUNK * N_CAS_STEPS,), jnp.int32),
        pltpu.VMEM((CHUNK * N_USER_STEPS,), jnp.int32),
        pltpu.VMEM((CHUNK, 16), jnp.float32),
        pltpu.VMEM((CHUNK * N_CAS_STEPS, DIM), jnp.float32),
        pltpu.VMEM((CHUNK * N_USER_STEPS, DIM), jnp.float32),
        pltpu.VMEM((CHUNK, DIM), jnp.float32),
        pltpu.VMEM((DIM,), jnp.float32),
        pltpu.SemaphoreType.DMA,
        pltpu.SemaphoreType.DMA,
    ],
)(_sc_body)


def kernel(walk_nodes, predict_times, user_table, cas_table, time_w):
    wn = walk_nodes.reshape(PATHS, WALK_LEN)
    cas_idx = wn[:, 1::2].reshape(NUM_WORKERS, NUM_CHUNKS,
                                  CHUNK * N_CAS_STEPS)
    user_idx = wn[:, 2::2].reshape(NUM_WORKERS, NUM_CHUNKS,
                                   CHUNK * N_USER_STEPS)
    pt = jnp.broadcast_to(predict_times[:, None, None],
                          (BATCH, WALK_NUM, 16))
    pt = pt.reshape(NUM_WORKERS, NUM_CHUNKS, CHUNK, 16)
    return _sc_call(cas_idx, user_idx, pt, cas_table, user_table, time_w)


# speedup vs baseline: 1.0877x; 1.0877x over previous
import functools

import jax
import jax.numpy as jnp
from jax import lax
from jax.experimental import pallas as pl
from jax.experimental.pallas import tpu as pltpu
from jax.experimental.pallas import tpu_sc as plsc

BATCH = 4096
WALK_NUM = 8
WALK_LEN = 8
DIM = 64
PATHS = BATCH * WALK_NUM

NUM_CORES = 2
NUM_SUBCORES = 16
NUM_WORKERS = NUM_CORES * NUM_SUBCORES
PATHS_PER_WORKER = PATHS // NUM_WORKERS
CHUNK = 32
NUM_CHUNKS = PATHS_PER_WORKER // CHUNK

N_CAS = 4
N_USER = 3
INV_STEPS = 1.0 / (N_CAS + N_USER)
NVR = DIM // 16


def _sc_body(cas_idx_h, user_idx_h, pt_h, cas_tab_h, user_tab_h, tw_h,
             out_h,
             cas_idx_v, user_idx_v, pt_v, tw_v,
             cas_rows, user_rows, out_v,
             sem_c0, sem_u0, sem_c1, sem_u1):
    wid = lax.axis_index("s") * NUM_CORES + lax.axis_index("c")

    pltpu.sync_copy(cas_idx_h.at[wid], cas_idx_v)
    pltpu.sync_copy(user_idx_h.at[wid], user_idx_v)
    pltpu.sync_copy(pt_h.at[wid], pt_v)
    pltpu.sync_copy(tw_h, tw_v)
    tw4 = [tw_v[pl.ds(16 * d, 16)] * (N_CAS * INV_STEPS)
           for d in range(NVR)]

    sem_c = [sem_c0, sem_c1]
    sem_u = [sem_u0, sem_u1]

    def fire(k, buf):
        pltpu.async_copy(cas_tab_h.at[cas_idx_v.at[k]], cas_rows.at[buf],
                         sem_c[buf])
        pltpu.async_copy(user_tab_h.at[user_idx_v.at[k]], user_rows.at[buf],
                         sem_u[buf])

    def wait(k, buf):
        pltpu.make_async_copy(cas_tab_h.at[cas_idx_v.at[k]],
                              cas_rows.at[buf], sem_c[buf]).wait()
        pltpu.make_async_copy(user_tab_h.at[user_idx_v.at[k]],
                              user_rows.at[buf], sem_u[buf]).wait()

    def compute(k, buf):
        cr = cas_rows.at[buf]
        ur = user_rows.at[buf]
        for c in range(CHUNK):
            ptv = pt_v[k * CHUNK + c, :]
            for d in range(NVR):
                sl = pl.ds(16 * d, 16)
                acc = cr[c * N_CAS + 0, sl]
                acc = acc + cr[c * N_CAS + 1, sl]
                acc = acc + cr[c * N_CAS + 2, sl]
                acc = acc + cr[c * N_CAS + 3, sl]
                acc = acc + ur[c * N_USER + 0, sl]
                acc = acc + ur[c * N_USER + 1, sl]
                acc = acc + ur[c * N_USER + 2, sl]
                out_v[c, sl] = acc * INV_STEPS + ptv * tw4[d]
        pltpu.sync_copy(
            out_v, out_h.at[pl.ds(wid * PATHS_PER_WORKER + k * CHUNK, CHUNK)])

    fire(0, 0)

    def pair_step(kk, carry):
        k0 = 2 * kk
        k1 = k0 + 1
        fire(k1, 1)
        wait(k0, 0)
        compute(k0, 0)

        @pl.when(kk < NUM_CHUNKS // 2 - 1)
        def _():
            fire(k0 + 2, 0)

        wait(k1, 1)
        compute(k1, 1)
        return carry

    lax.fori_loop(0, NUM_CHUNKS // 2, pair_step, 0)


_sc_call = functools.partial(
    pl.kernel,
    out_type=jax.ShapeDtypeStruct((PATHS, DIM), jnp.float32),
    mesh=plsc.VectorSubcoreMesh(core_axis_name="c", subcore_axis_name="s",
                                num_cores=NUM_CORES,
                                num_subcores=NUM_SUBCORES),
    compiler_params=pltpu.CompilerParams(use_tc_tiling_on_sc=False),
    scratch_types=[
        pltpu.VMEM((NUM_CHUNKS, CHUNK * N_CAS), jnp.int32),
        pltpu.VMEM((NUM_CHUNKS, CHUNK * N_USER), jnp.int32),
        pltpu.VMEM((PATHS_PER_WORKER, 16), jnp.float32),
        pltpu.VMEM((DIM,), jnp.float32),
        pltpu.VMEM((2, CHUNK * N_CAS, DIM), jnp.float32),
        pltpu.VMEM((2, CHUNK * N_USER, DIM), jnp.float32),
        pltpu.VMEM((CHUNK, DIM), jnp.float32),
        pltpu.SemaphoreType.DMA,
        pltpu.SemaphoreType.DMA,
        pltpu.SemaphoreType.DMA,
        pltpu.SemaphoreType.DMA,
    ],
)(_sc_body)


def kernel(walk_nodes, predict_times, user_table, cas_table, time_w):
    wn = walk_nodes.reshape(PATHS, WALK_LEN)
    cas_idx = wn[:, 1::2].reshape(NUM_WORKERS, NUM_CHUNKS, CHUNK * N_CAS)
    user_idx = wn[:, 2::2].reshape(NUM_WORKERS, NUM_CHUNKS, CHUNK * N_USER)
    pt = jnp.broadcast_to(predict_times[:, None, None],
                          (BATCH, WALK_NUM, 16))
    pt = pt.reshape(NUM_WORKERS, PATHS_PER_WORKER, 16)
    return _sc_call(cas_idx, user_idx, pt, cas_table, user_table, time_w)


# speedup vs baseline: 1.1189x; 1.0287x over previous
import functools

import jax
import jax.numpy as jnp
from jax import lax
from jax.experimental import pallas as pl
from jax.experimental.pallas import tpu as pltpu
from jax.experimental.pallas import tpu_sc as plsc

BATCH = 4096
WALK_NUM = 8
WALK_LEN = 8
DIM = 64
PATHS = BATCH * WALK_NUM

NUM_CORES = 2
NUM_SUBCORES = 16
NUM_WORKERS = NUM_CORES * NUM_SUBCORES
PPW = PATHS // NUM_WORKERS
CHUNK = 32
NUM_CHUNKS = PPW // CHUNK

N_CAS = 4
N_USER = 3
INV_STEPS = 1.0 / (N_CAS + N_USER)
NVR = DIM // 16
BPW = BATCH // NUM_WORKERS


def _sc_body(wn_h, pt_h, cas_tab_h, user_tab_h, tw_h,
             out_h,
             slab_v, pts_v, cas_idx_v, user_idx_v, pt_v, tw_v,
             cas_rows, user_rows, out_v,
             sem_c0, sem_u0, sem_c1, sem_u1):
    wid = lax.axis_index("s") * NUM_CORES + lax.axis_index("c")

    pltpu.sync_copy(wn_h.at[pl.ds(wid * PPW * WALK_LEN, PPW * WALK_LEN)],
                    slab_v)
    pltpu.sync_copy(pt_h.at[pl.ds(wid * BPW, BPW)], pts_v)
    pltpu.sync_copy(tw_h, tw_v)
    tw4 = [tw_v[pl.ds(16 * d, 16)] * (N_CAS * INV_STEPS)
           for d in range(NVR)]

    lanes = lax.iota(jnp.int32, 16)
    pat_cas = ((lanes >> 2) << 3) + ((lanes & 3) << 1) + 1
    pat_user = []
    for v in range(3):
        u = lanes + (16 * v)
        pat_user.append((u // 3) * 8 + (u % 3) * 2 + 2)

    def build_cas(g, carry):
        vec = plsc.load_gather(slab_v, [pat_cas + g * 32])
        cas_idx_v[pl.ds(g * 16, 16)] = vec
        return carry

    lax.fori_loop(0, PPW // 4, build_cas, 0)

    def build_user(g, carry):
        for v in range(3):
            vec = plsc.load_gather(slab_v, [pat_user[v] + g * 128])
            user_idx_v[pl.ds(g * 48 + v * 16, 16)] = vec
        return carry

    lax.fori_loop(0, PPW // 16, build_user, 0)

    def build_pt(b, carry):
        pt_v[b, :] = plsc.load_gather(pts_v, [jnp.zeros((16,), jnp.int32) + b])
        return carry

    lax.fori_loop(0, BPW, build_pt, 0)

    sem_c = [sem_c0, sem_c1]
    sem_u = [sem_u0, sem_u1]

    def fire(k, buf):
        pltpu.async_copy(cas_tab_h.at[cas_idx_v.at[pl.ds(k * 128, 128)]],
                         cas_rows.at[buf], sem_c[buf])
        pltpu.async_copy(user_tab_h.at[user_idx_v.at[pl.ds(k * 96, 96)]],
                         user_rows.at[buf], sem_u[buf])

    def wait(k, buf):
        pltpu.make_async_copy(cas_tab_h.at[cas_idx_v.at[pl.ds(k * 128, 128)]],
                              cas_rows.at[buf], sem_c[buf]).wait()
        pltpu.make_async_copy(user_tab_h.at[user_idx_v.at[pl.ds(k * 96, 96)]],
                              user_rows.at[buf], sem_u[buf]).wait()

    def compute(k, buf):
        cr = cas_rows.at[buf]
        ur = user_rows.at[buf]
        for c in range(CHUNK):
            ptv = pt_v[(k * CHUNK + c) >> 3, :]
            for d in range(NVR):
                sl = pl.ds(16 * d, 16)
                acc = cr[c * N_CAS + 0, sl]
                acc = acc + cr[c * N_CAS + 1, sl]
                acc = acc + cr[c * N_CAS + 2, sl]
                acc = acc + cr[c * N_CAS + 3, sl]
                acc = acc + ur[c * N_USER + 0, sl]
                acc = acc + ur[c * N_USER + 1, sl]
                acc = acc + ur[c * N_USER + 2, sl]
                out_v[c, sl] = acc * INV_STEPS + ptv * tw4[d]
        pltpu.sync_copy(out_v, out_h.at[pl.ds(wid * PPW + k * CHUNK, CHUNK)])

    fire(0, 0)

    def pair_step(kk, carry):
        k0 = 2 * kk
        k1 = k0 + 1
        fire(k1, 1)
        wait(k0, 0)
        compute(k0, 0)

        @pl.when(kk < NUM_CHUNKS // 2 - 1)
        def _():
            fire(k0 + 2, 0)

        wait(k1, 1)
        compute(k1, 1)
        return carry

    lax.fori_loop(0, NUM_CHUNKS // 2, pair_step, 0)


_sc_call = functools.partial(
    pl.kernel,
    out_type=jax.ShapeDtypeStruct((PATHS, DIM), jnp.float32),
    mesh=plsc.VectorSubcoreMesh(core_axis_name="c", subcore_axis_name="s",
                                num_cores=NUM_CORES,
                                num_subcores=NUM_SUBCORES),
    compiler_params=pltpu.CompilerParams(use_tc_tiling_on_sc=False,
                                         needs_layout_passes=False),
    scratch_types=[
        pltpu.VMEM((PPW * WALK_LEN,), jnp.int32),
        pltpu.VMEM((BPW,), jnp.float32),
        pltpu.VMEM((PPW * N_CAS,), jnp.int32),
        pltpu.VMEM((PPW * N_USER,), jnp.int32),
        pltpu.VMEM((BPW, 16), jnp.float32),
        pltpu.VMEM((DIM,), jnp.float32),
        pltpu.VMEM((2, CHUNK * N_CAS, DIM), jnp.float32),
        pltpu.VMEM((2, CHUNK * N_USER, DIM), jnp.float32),
        pltpu.VMEM((CHUNK, DIM), jnp.float32),
        pltpu.SemaphoreType.DMA,
        pltpu.SemaphoreType.DMA,
        pltpu.SemaphoreType.DMA,
        pltpu.SemaphoreType.DMA,
    ],
)(_sc_body)


def kernel(walk_nodes, predict_times, user_table, cas_table, time_w):
    wn = walk_nodes.reshape(PATHS * WALK_LEN)
    return _sc_call(wn, predict_times, cas_table, user_table, time_w)


# speedup vs baseline: 1.9475x; 1.7406x over previous
import functools

import jax
import jax.numpy as jnp
from jax import lax
from jax.experimental import pallas as pl
from jax.experimental.pallas import tpu as pltpu
from jax.experimental.pallas import tpu_sc as plsc

BATCH = 4096
WALK_NUM = 8
WALK_LEN = 8
DIM = 64
PATHS = BATCH * WALK_NUM

NUM_CORES = 2
NUM_SUBCORES = 16
NUM_WORKERS = NUM_CORES * NUM_SUBCORES
PPW = PATHS // NUM_WORKERS
CHUNK = 32
NUM_CHUNKS = PPW // CHUNK

CAS_STEPS = 4
USER_STEPS = 3
INV_STEPS = 1.0 / (CAS_STEPS + USER_STEPS)
NVR = DIM // 16
BPW = BATCH // NUM_WORKERS

TBLK = 8192

_SC_PARAMS = pltpu.CompilerParams(use_tc_tiling_on_sc=False,
                                  needs_layout_passes=False)
_SC_MESH = plsc.VectorSubcoreMesh(core_axis_name="c", subcore_axis_name="s",
                                  num_cores=NUM_CORES,
                                  num_subcores=NUM_SUBCORES)


def _tr_body(src_ref, out_ref):
    t = src_ref[...].T
    out_ref[...] = jnp.concatenate(
        [t, jnp.zeros((TBLK, 128 - DIM), jnp.float32)], axis=1)


def _transpose_table(table_t):
    rows = table_t.shape[1]
    grid = (rows + TBLK - 1) // TBLK
    return pl.pallas_call(
        _tr_body,
        grid=(grid,),
        in_specs=[pl.BlockSpec((DIM, TBLK), lambda j: (0, j))],
        out_specs=pl.BlockSpec((TBLK, 128), lambda j: (j, 0)),
        out_shape=jax.ShapeDtypeStruct((rows, 128), jnp.float32),
    )(table_t)


def _stage_slab(wn_h, wid, slab_v):
    pltpu.sync_copy(wn_h.at[pl.ds(wid * PPW * WALK_LEN, PPW * WALK_LEN)],
                    slab_v)


def _cas_body(wn_h, pt_h, cas_tab_h, tw_h,
              out_h,
              slab_v, pts_v, cas_idx_v, pt_v, tw_v, cas_rows, out_v,
              sem0, sem1):
    wid = lax.axis_index("s") * NUM_CORES + lax.axis_index("c")
    _stage_slab(wn_h, wid, slab_v)
    pltpu.sync_copy(pt_h.at[pl.ds(wid * BPW, BPW)], pts_v)
    pltpu.sync_copy(tw_h, tw_v)
    tw4 = [tw_v[pl.ds(16 * d, 16)] * (CAS_STEPS * INV_STEPS)
           for d in range(NVR)]

    lanes = lax.iota(jnp.int32, 16)
    pat_cas = ((lanes >> 2) << 3) + ((lanes & 3) << 1) + 1

    def build_cas(g, carry):
        cas_idx_v[pl.ds(g * 16, 16)] = plsc.load_gather(
            slab_v, [pat_cas + g * 32])
        return carry

    lax.fori_loop(0, PPW // 4, build_cas, 0)

    def build_pt(b, carry):
        pt_v[b, :] = plsc.load_gather(pts_v, [jnp.zeros((16,), jnp.int32) + b])
        return carry

    lax.fori_loop(0, BPW, build_pt, 0)

    sems = [sem0, sem1]

    def fire(k, buf):
        pltpu.async_copy(cas_tab_h.at[cas_idx_v.at[pl.ds(k * 128, 128)]],
                         cas_rows.at[buf], sems[buf])

    def wait(k, buf):
        pltpu.make_async_copy(cas_tab_h.at[cas_idx_v.at[pl.ds(k * 128, 128)]],
                              cas_rows.at[buf], sems[buf]).wait()

    def compute(k, buf):
        cr = cas_rows.at[buf]
        for c in range(CHUNK):
            ptv = pt_v[(k * CHUNK + c) >> 3, :]
            for d in range(NVR):
                sl = pl.ds(16 * d, 16)
                acc = cr[c * CAS_STEPS + 0, sl]
                acc = acc + cr[c * CAS_STEPS + 1, sl]
                acc = acc + cr[c * CAS_STEPS + 2, sl]
                acc = acc + cr[c * CAS_STEPS + 3, sl]
                out_v[c, sl] = acc * INV_STEPS + ptv * tw4[d]
        pltpu.sync_copy(out_v, out_h.at[pl.ds(wid * PPW + k * CHUNK, CHUNK)])

    fire(0, 0)

    def pair_step(kk, carry):
        k0 = 2 * kk
        k1 = k0 + 1
        fire(k1, 1)
        wait(k0, 0)
        compute(k0, 0)

        @pl.when(kk < NUM_CHUNKS // 2 - 1)
        def _():
            fire(k0 + 2, 0)

        wait(k1, 1)
        compute(k1, 1)
        return carry

    lax.fori_loop(0, NUM_CHUNKS // 2, pair_step, 0)


def _user_body(wn_h, user_tab_h, part_h,
               out_h,
               slab_v, user_idx_v, user_rows, part_v, out_v,
               sem0, sem1, semp):
    wid = lax.axis_index("s") * NUM_CORES + lax.axis_index("c")
    _stage_slab(wn_h, wid, slab_v)

    lanes = lax.iota(jnp.int32, 16)
    pat_user = []
    for v in range(3):
        u = lanes + (16 * v)
        pat_user.append((u // 3) * 8 + (u % 3) * 2 + 2)

    def build_user(g, carry):
        for v in range(3):
            user_idx_v[pl.ds(g * 48 + v * 16, 16)] = plsc.load_gather(
                slab_v, [pat_user[v] + g * 128])
        return carry

    lax.fori_loop(0, PPW // 16, build_user, 0)

    sems = [sem0, sem1]

    def fire(k, buf):
        pltpu.async_copy(user_tab_h.at[user_idx_v.at[pl.ds(k * 96, 96)]],
                         user_rows.at[buf], sems[buf])
        pltpu.async_copy(part_h.at[pl.ds(wid * PPW + k * CHUNK, CHUNK)],
                         part_v.at[buf], semp)

    def wait(k, buf):
        pltpu.make_async_copy(user_tab_h.at[user_idx_v.at[pl.ds(k * 96, 96)]],
                              user_rows.at[buf], sems[buf]).wait()
        pltpu.make_async_copy(part_h.at[pl.ds(wid * PPW + k * CHUNK, CHUNK)],
                              part_v.at[buf], semp).wait()

    def compute(k, buf):
        ur = user_rows.at[buf]
        pv = part_v.at[buf]
        for c in range(CHUNK):
            for d in range(NVR):
                sl = pl.ds(16 * d, 16)
                acc = ur[c * USER_STEPS + 0, sl]
                acc = acc + ur[c * USER_STEPS + 1, sl]
                acc = acc + ur[c * USER_STEPS + 2, sl]
                out_v[c, sl] = pv[c, sl] + acc * INV_STEPS
        pltpu.sync_copy(out_v, out_h.at[pl.ds(wid * PPW + k * CHUNK, CHUNK)])

    fire(0, 0)

    def pair_step(kk, carry):
        k0 = 2 * kk
        k1 = k0 + 1
        fire(k1, 1)
        wait(k0, 0)
        compute(k0, 0)

        @pl.when(kk < NUM_CHUNKS // 2 - 1)
        def _():
            fire(k0 + 2, 0)

        wait(k1, 1)
        compute(k1, 1)
        return carry

    lax.fori_loop(0, NUM_CHUNKS // 2, pair_step, 0)


_cas_call = functools.partial(
    pl.kernel,
    out_type=jax.ShapeDtypeStruct((PATHS, DIM), jnp.float32),
    mesh=_SC_MESH,
    compiler_params=_SC_PARAMS,
    scratch_types=[
        pltpu.VMEM((PPW * WALK_LEN,), jnp.int32),
        pltpu.VMEM((BPW,), jnp.float32),
        pltpu.VMEM((PPW * CAS_STEPS,), jnp.int32),
        pltpu.VMEM((BPW, 16), jnp.float32),
        pltpu.VMEM((DIM,), jnp.float32),
        pltpu.VMEM((2, CHUNK * CAS_STEPS, 128), jnp.float32),
        pltpu.VMEM((CHUNK, DIM), jnp.float32),
        pltpu.SemaphoreType.DMA,
        pltpu.SemaphoreType.DMA,
    ],
)(_cas_body)

_user_call = functools.partial(
    pl.kernel,
    out_type=jax.ShapeDtypeStruct((PATHS, DIM), jnp.float32),
    mesh=_SC_MESH,
    compiler_params=_SC_PARAMS,
    scratch_types=[
        pltpu.VMEM((PPW * WALK_LEN,), jnp.int32),
        pltpu.VMEM((PPW * USER_STEPS,), jnp.int32),
        pltpu.VMEM((2, CHUNK * USER_STEPS, 128), jnp.float32),
        pltpu.VMEM((2, CHUNK, DIM), jnp.float32),
        pltpu.VMEM((CHUNK, DIM), jnp.float32),
        pltpu.SemaphoreType.DMA,
        pltpu.SemaphoreType.DMA,
        pltpu.SemaphoreType.DMA,
    ],
)(_user_body)


def kernel(walk_nodes, predict_times, user_table, cas_table, time_w):
    cas_lin = _transpose_table(jnp.swapaxes(cas_table, 0, 1))
    user_lin = _transpose_table(jnp.swapaxes(user_table, 0, 1))
    wn = walk_nodes.reshape(PATHS * WALK_LEN)
    partial = _cas_call(wn, predict_times, cas_lin, time_w)
    return _user_call(wn, user_lin, partial)
